# baseline (speedup 1.0000x reference)
import functools

import jax
import jax.numpy as jnp
from jax import lax
from jax.experimental import pallas as pl
from jax.experimental.pallas import tpu as pltpu
from jax.experimental.pallas import tpu_sc as plsc

FOLD_EMB_DIM = 128
N_SEQ = 512
BS = 256
BB = 8

_info = plsc.get_sparse_core_info()
_NC, _NS = _info.num_cores, _info.num_subcores
_NW = _NC * _NS
_BPW = BS // _NW


def _sc_gather(idx_hbm, c_hbm, a_hbm, t_hbm, out_hbm, idx_v, rows_v, sem):
    wid = lax.axis_index("s") * _NC + lax.axis_index("c")
    base = wid * _BPW
    for k, tbl in enumerate((c_hbm, a_hbm, t_hbm)):
        pltpu.sync_copy(idx_hbm.at[k, pl.ds(base, _BPW)], idx_v)
        pltpu.async_copy(tbl.at[idx_v], rows_v, sem).wait()
        pltpu.sync_copy(rows_v, out_hbm.at[k, pl.ds(base, _BPW)])


def _gather_feat(idx_t, emb_C, emb_A, emb_T):
    mesh = plsc.VectorSubcoreMesh(core_axis_name="c", subcore_axis_name="s")
    run = functools.partial(
        pl.kernel,
        mesh=mesh,
        out_type=jax.ShapeDtypeStruct((3, BS, FOLD_EMB_DIM), jnp.float32),
        scratch_types=[
            pltpu.VMEM((_BPW,), jnp.int32),
            pltpu.VMEM((_BPW, FOLD_EMB_DIM), jnp.float32),
            pltpu.SemaphoreType.DMA,
        ],
    )(_sc_gather)
    return run(idx_t, emb_C, emb_A, emb_T)


def _bcast_body(feat_ref, out_ref):
    for r in range(BB):
        for k in range(3):
            row = feat_ref[k, r, :]
            out_ref[r, :, 128 * k:128 * (k + 1)] = jnp.broadcast_to(
                row[None, :], (N_SEQ, 128))


def kernel(cath_idx, n, emb_C, emb_A, emb_T):
    del n
    bs = cath_idx.shape[0]
    idx_t = cath_idx.astype(jnp.int32).T
    feat3 = _gather_feat(idx_t, emb_C, emb_A, emb_T)
    d = 3 * FOLD_EMB_DIM
    out = pl.pallas_call(
        _bcast_body,
        grid=(bs // BB,),
        in_specs=[pl.BlockSpec((3, BB, FOLD_EMB_DIM), lambda i: (0, i, 0))],
        out_specs=pl.BlockSpec((BB, N_SEQ, d), lambda i: (i, 0, 0)),
        out_shape=jax.ShapeDtypeStruct((bs, N_SEQ, d), jnp.float32),
    )(feat3)
    return out


# speedup vs baseline: 1.0348x; 1.0348x over previous
import functools

import jax
import jax.numpy as jnp
from jax import lax
from jax.experimental import pallas as pl
from jax.experimental.pallas import tpu as pltpu
from jax.experimental.pallas import tpu_sc as plsc

FOLD_EMB_DIM = 128
N_SEQ = 512
BS = 256
BB = 8

_info = plsc.get_sparse_core_info()
_NC, _NS = _info.num_cores, _info.num_subcores
_NW = _NC * _NS
_BPW = BS // _NW


def _sc_gather(idx_hbm, c_hbm, a_hbm, t_hbm, out_hbm, idx_v, rows_v, sem):
    wid = lax.axis_index("s") * _NC + lax.axis_index("c")
    pltpu.sync_copy(idx_hbm.at[wid], idx_v)
    copies = []
    for k, tbl in enumerate((c_hbm, a_hbm, t_hbm)):
        copies.append(pltpu.async_copy(tbl.at[idx_v.at[k]], rows_v.at[k], sem))
    for c in copies:
        c.wait()
    pltpu.sync_copy(rows_v, out_hbm.at[wid])


def _gather_feat(idx_w, emb_C, emb_A, emb_T):
    mesh = plsc.VectorSubcoreMesh(core_axis_name="c", subcore_axis_name="s")
    run = functools.partial(
        pl.kernel,
        mesh=mesh,
        out_type=jax.ShapeDtypeStruct((_NW, 3, _BPW, FOLD_EMB_DIM),
                                      jnp.float32),
        scratch_types=[
            pltpu.VMEM((3, _BPW), jnp.int32),
            pltpu.VMEM((3, _BPW, FOLD_EMB_DIM), jnp.float32),
            pltpu.SemaphoreType.DMA,
        ],
    )(_sc_gather)
    return run(idx_w, emb_C, emb_A, emb_T)


def _bcast_body(feat_ref, out_ref):
    for r in range(BB):
        for k in range(3):
            row = feat_ref[0, k, r, :]
            out_ref[r, :, 128 * k:128 * (k + 1)] = jnp.broadcast_to(
                row[None, :], (N_SEQ, 128))


def kernel(cath_idx, n, emb_C, emb_A, emb_T):
    del n
    bs = cath_idx.shape[0]
    idx_w = (cath_idx.astype(jnp.int32).T
             .reshape(3, _NW, _BPW).transpose(1, 0, 2))
    feat4 = _gather_feat(idx_w, emb_C, emb_A, emb_T)
    d = 3 * FOLD_EMB_DIM
    out = pl.pallas_call(
        _bcast_body,
        grid=(bs // BB,),
        in_specs=[
            pl.BlockSpec((1, 3, _BPW, FOLD_EMB_DIM), lambda i: (i, 0, 0, 0))
        ],
        out_specs=pl.BlockSpec((BB, N_SEQ, d), lambda i: (i, 0, 0)),
        out_shape=jax.ShapeDtypeStruct((bs, N_SEQ, d), jnp.float32),
    )(feat4)
    return out
